# baseline (speedup 1.0000x reference)
import jax
import jax.numpy as jnp
from jax.experimental import pallas as pl


def _conv(x, W, a_s, a_d, src, dst, et):
    h = jnp.einsum('nd,rdo->rno', x, W)
    h_src = h[et, src]
    h_dst = h[et, dst]
    e = jnp.sum(h_src * a_s[et], axis=-1) + jnp.sum(h_dst * a_d[et], axis=-1)
    e = jax.nn.leaky_relu(e, negative_slope=0.2)
    m = jax.ops.segment_max(e, dst, num_segments=10000)
    m = jnp.where(jnp.isfinite(m), m, 0.0)
    ex = jnp.exp(e - jax.lax.stop_gradient(m)[dst])
    denom = jax.ops.segment_sum(ex, dst, num_segments=10000)
    alpha = ex / (denom[dst] + 1e-16)
    return jax.ops.segment_sum(alpha[:, None] * h_src, dst, num_segments=10000)


def _identity_kernel(x_ref, o_ref):
    o_ref[...] = x_ref[...]


def kernel(x, edge_index, edge_type, W0, a_src0, a_dst0, W1, a_src1, a_dst1, W2, a_src2, a_dst2):
    src = edge_index[0]
    dst = edge_index[1]
    h = jax.nn.relu(_conv(x, W0, a_src0, a_dst0, src, dst, edge_type))
    h = jax.nn.relu(_conv(h, W1, a_src1, a_dst1, src, dst, edge_type))
    out = _conv(h, W2, a_src2, a_dst2, src, dst, edge_type)
    out = pl.pallas_call(
        _identity_kernel,
        out_shape=jax.ShapeDtypeStruct(out.shape, out.dtype),
    )(out)
    return out


# speedup vs baseline: 8.6868x; 8.6868x over previous
import functools

import jax
import jax.numpy as jnp
from jax import lax
from jax.experimental import pallas as pl
from jax.experimental.pallas import tpu as pltpu
from jax.experimental.pallas import tpu_sc as plsc

N = 10000
E = 160000
R = 8
D = 128

NC = 2
NS = 16
NW = NC * NS
K = 128
EW = 5120
NCH = EW // K
EP = NW * EW
NSH = 10240
ROWS_PER_SUB = NSH // NS

TN = 400
NB = N // TN


def _prep_body(s_ref, d_ref, t_ref, ih_ref, is_ref, id_ref):
    s = s_ref[...]
    d = d_ref[...]
    t = t_ref[...]
    ih_ref[...] = t * N + s
    is_ref[...] = s * R + t
    id_ref[...] = jnp.minimum(d, N - 1) * R + t


def _prep_indices(src_p, dst_p, et_p):
    shp = (EP // 128, 128)
    out = pl.pallas_call(
        _prep_body,
        out_shape=[jax.ShapeDtypeStruct(shp, jnp.int32)] * 3,
    )(src_p.reshape(shp), dst_p.reshape(shp), et_p.reshape(shp))
    return [o.reshape(EP) for o in out]


def _dense_body(x_ref, w_ref, bs_ref, bd_ref,
                h_ref, ss_ref, sd_ref, rm_ref):
    xb = x_ref[...]
    h_ref[0] = jnp.dot(xb, w_ref[0], preferred_element_type=jnp.float32)

    @pl.when(pl.program_id(1) == 0)
    def _():
        ss = jnp.dot(xb, bs_ref[...], preferred_element_type=jnp.float32)
        sd = jnp.dot(xb, bd_ref[...], preferred_element_type=jnp.float32)
        ss_ref[...] = ss
        sd_ref[...] = sd
        rm_ref[...] = jnp.max(sd, axis=1, keepdims=True)


def _dense(x, W, Bs, Bd):
    return pl.pallas_call(
        _dense_body,
        grid=(NB, R),
        in_specs=[
            pl.BlockSpec((TN, D), lambda n, r: (n, 0)),
            pl.BlockSpec((1, D, D), lambda n, r: (r, 0, 0)),
            pl.BlockSpec((D, R), lambda n, r: (0, 0)),
            pl.BlockSpec((D, R), lambda n, r: (0, 0)),
        ],
        out_specs=[
            pl.BlockSpec((1, TN, D), lambda n, r: (r, n, 0)),
            pl.BlockSpec((TN, R), lambda n, r: (n, 0)),
            pl.BlockSpec((TN, R), lambda n, r: (n, 0)),
            pl.BlockSpec((TN, 1), lambda n, r: (n, 0)),
        ],
        out_shape=[
            jax.ShapeDtypeStruct((R, N, D), jnp.float32),
            jax.ShapeDtypeStruct((N, R), jnp.float32),
            jax.ShapeDtypeStruct((N, R), jnp.float32),
            jax.ShapeDtypeStruct((N, 1), jnp.float32),
        ],
    )(x, W, Bs, Bd)


def _norm_body(o0_ref, o1_ref, d0_ref, d1_ref, x_ref, *, relu):
    o = o0_ref[0] + o1_ref[0]
    d = d0_ref[0] + d1_ref[0]
    x = o * (1.0 / (d + 1e-16))
    if relu:
        x = jnp.maximum(x, 0.0)
    x_ref[...] = x


def _normalize(outp, denp, relu):
    denp3 = denp.reshape(NC, NSH, 1)
    return pl.pallas_call(
        functools.partial(_norm_body, relu=relu),
        grid=(NB,),
        in_specs=[
            pl.BlockSpec((1, TN, D), lambda n: (0, n, 0)),
            pl.BlockSpec((1, TN, D), lambda n: (1, n, 0)),
            pl.BlockSpec((1, TN, 1), lambda n: (0, n, 0)),
            pl.BlockSpec((1, TN, 1), lambda n: (1, n, 0)),
        ],
        out_specs=pl.BlockSpec((TN, D), lambda n: (n, 0)),
        out_shape=jax.ShapeDtypeStruct((N, D), jnp.float32),
    )(outp, outp, denp3, denp3)


def _sc_body(ih_hbm, is_hbm, id_hbm, dst_hbm, ssrc_hbm, sdst_hbm, cmax_hbm,
             h_hbm, outp_hbm, denp_hbm,
             ih_v, is_v, id_v, dst_v, es_v, ed_v, cb_v, ex_v, rows_v, zden_v,
             out_sh, den_sh):
    cid = lax.axis_index("c")
    sid = lax.axis_index("s")
    wid = sid * NC + cid

    @pl.loop(0, K)
    def _(i):
        for j in range(D // 16):
            rows_v[i, pl.ds(j * 16, 16)] = jnp.zeros((16,), jnp.float32)

    @pl.loop(0, 1280 // 16)
    def _(i):
        zden_v[pl.ds(i * 16, 16)] = jnp.zeros((16,), jnp.float32)

    for z in range(ROWS_PER_SUB // K):
        pltpu.sync_copy(rows_v, out_sh.at[pl.ds(sid * ROWS_PER_SUB + z * K, K)])

    @pl.when(sid == 0)
    def _():
        for z in range(NSH // 1280):
            pltpu.sync_copy(zden_v, den_sh.at[pl.ds(z * 1280, 1280)])

    plsc.subcore_barrier()

    @pl.loop(0, NCH)
    def _(ci):
        base = pl.multiple_of(wid * EW + ci * K, K)
        pltpu.sync_copy(ih_hbm.at[pl.ds(base, K)], ih_v)
        pltpu.sync_copy(is_hbm.at[pl.ds(base, K)], is_v)
        pltpu.sync_copy(id_hbm.at[pl.ds(base, K)], id_v)
        pltpu.sync_copy(dst_hbm.at[pl.ds(base, K)], dst_v)
        pltpu.sync_copy(ssrc_hbm.at[is_v], es_v)
        pltpu.sync_copy(sdst_hbm.at[id_v], ed_v)
        pltpu.sync_copy(cmax_hbm.at[dst_v], cb_v)
        pltpu.sync_copy(h_hbm.at[ih_v], rows_v)

        @pl.loop(0, K // 16)
        def _(g):
            sl = pl.ds(g * 16, 16)
            e = es_v[sl] + ed_v[sl]
            e = jnp.where(e >= 0.0, e, e * 0.2)
            ex_v[sl] = jnp.exp(e - cb_v[sl])

        @pl.loop(0, K)
        def _(i):
            sv = plsc.load_gather(ex_v, [jnp.full((16,), i, jnp.int32)])
            for j in range(D // 16):
                sl = pl.ds(j * 16, 16)
                rows_v[i, sl] = rows_v[i, sl] * sv

        pltpu.sync_copy(rows_v, out_sh.at[dst_v], add=True)
        pltpu.sync_copy(ex_v, den_sh.at[dst_v], add=True)

    plsc.subcore_barrier()

    for z in range(ROWS_PER_SUB // K):
        off = sid * ROWS_PER_SUB + z * K
        pltpu.sync_copy(out_sh.at[pl.ds(off, K)], outp_hbm.at[cid, pl.ds(off, K)])

    @pl.when(sid == 0)
    def _():
        pltpu.sync_copy(den_sh, denp_hbm.at[cid])


_sc_edge = pl.kernel(
    _sc_body,
    out_type=[
        jax.ShapeDtypeStruct((NC, NSH, D), jnp.float32),
        jax.ShapeDtypeStruct((NC, NSH), jnp.float32),
    ],
    mesh=plsc.VectorSubcoreMesh(core_axis_name="c", subcore_axis_name="s"),
    compiler_params=pltpu.CompilerParams(needs_layout_passes=False),
    scratch_types=[
        pltpu.VMEM((K,), jnp.int32),
        pltpu.VMEM((K,), jnp.int32),
        pltpu.VMEM((K,), jnp.int32),
        pltpu.VMEM((K,), jnp.int32),
        pltpu.VMEM((K,), jnp.float32),
        pltpu.VMEM((K,), jnp.float32),
        pltpu.VMEM((K,), jnp.float32),
        pltpu.VMEM((K,), jnp.float32),
        pltpu.VMEM((K, D), jnp.float32),
        pltpu.VMEM((1280,), jnp.float32),
        pltpu.VMEM_SHARED((NSH, D), jnp.float32),
        pltpu.VMEM_SHARED((NSH,), jnp.float32),
    ],
)


def _layer(x, W, a_s, a_d, idx_h, idx_s, idx_d, dst_p):
    Bs = jnp.einsum('rdo,ro->dr', W, a_s)
    Bd = jnp.einsum('rdo,ro->dr', W, a_d)
    h, ssrc, sdst, rowmax = _dense(x, W, Bs, Bd)
    m_src = jnp.max(ssrc)
    cmax = jnp.concatenate(
        [jnp.maximum(rowmax[:, 0] + m_src, 0.0),
         jnp.full((1,), 1e30, jnp.float32)])
    outp, denp = _sc_edge(
        idx_h, idx_s, idx_d, dst_p,
        ssrc.reshape(N * R), sdst.reshape(N * R), cmax,
        h.reshape(R * N, D))
    return outp, denp


def kernel(x, edge_index, edge_type, W0, a_src0, a_dst0, W1, a_src1, a_dst1,
           W2, a_src2, a_dst2):
    src = edge_index[0]
    dst = edge_index[1]
    pad = EP - E
    src_p = jnp.concatenate([src, jnp.zeros((pad,), jnp.int32)])
    dst_p = jnp.concatenate([dst, jnp.full((pad,), N, jnp.int32)])
    et_p = jnp.concatenate([edge_type, jnp.zeros((pad,), jnp.int32)])
    idx_h, idx_s, idx_d = _prep_indices(src_p, dst_p, et_p)

    outp, denp = _layer(x, W0, a_src0, a_dst0, idx_h, idx_s, idx_d, dst_p)
    x1 = _normalize(outp, denp, relu=True)
    outp, denp = _layer(x1, W1, a_src1, a_dst1, idx_h, idx_s, idx_d, dst_p)
    x2 = _normalize(outp, denp, relu=True)
    outp, denp = _layer(x2, W2, a_src2, a_dst2, idx_h, idx_s, idx_d, dst_p)
    return _normalize(outp, denp, relu=False)


# speedup vs baseline: 13.4795x; 1.5517x over previous
import functools

import jax
import jax.numpy as jnp
from jax import lax
from jax.experimental import pallas as pl
from jax.experimental.pallas import tpu as pltpu
from jax.experimental.pallas import tpu_sc as plsc

N = 10000
E = 160000
R = 8
D = 128

NC = 2
NS = 16
NW = NC * NS
K = 128
EW = 5120
NCH = EW // K
EP = NW * EW
NSH = 10240
ROWS_PER_SUB = NSH // NS

TN = 400
NB = N // TN


def _prep_body(s_ref, d_ref, t_ref, ih_ref, is_ref, id_ref):
    s = s_ref[...]
    d = d_ref[...]
    t = t_ref[...]
    ih_ref[...] = t * N + s
    is_ref[...] = s * R + t
    id_ref[...] = jnp.minimum(d, N - 1) * R + t


def _prep_indices(src_p, dst_p, et_p):
    shp = (EP // 128, 128)
    out = pl.pallas_call(
        _prep_body,
        out_shape=[jax.ShapeDtypeStruct(shp, jnp.int32)] * 3,
    )(src_p.reshape(shp), dst_p.reshape(shp), et_p.reshape(shp))
    return [o.reshape(EP) for o in out]


def _dense_body(x_ref, w_ref, bs_ref, bd_ref,
                h_ref, ss_ref, sd_ref, rm_ref):
    xb = x_ref[...]
    h_ref[0] = jnp.dot(xb, w_ref[0], preferred_element_type=jnp.float32)

    @pl.when(pl.program_id(1) == 0)
    def _():
        ss = jnp.dot(xb, bs_ref[...], preferred_element_type=jnp.float32)
        sd = jnp.dot(xb, bd_ref[...], preferred_element_type=jnp.float32)
        ss_ref[...] = ss
        sd_ref[...] = sd
        rm_ref[...] = jnp.max(sd, axis=1, keepdims=True)


def _dense(x, W, Bs, Bd):
    return pl.pallas_call(
        _dense_body,
        grid=(NB, R),
        in_specs=[
            pl.BlockSpec((TN, D), lambda n, r: (n, 0)),
            pl.BlockSpec((1, D, D), lambda n, r: (r, 0, 0)),
            pl.BlockSpec((D, R), lambda n, r: (0, 0)),
            pl.BlockSpec((D, R), lambda n, r: (0, 0)),
        ],
        out_specs=[
            pl.BlockSpec((1, TN, D), lambda n, r: (r, n, 0)),
            pl.BlockSpec((TN, R), lambda n, r: (n, 0)),
            pl.BlockSpec((TN, R), lambda n, r: (n, 0)),
            pl.BlockSpec((TN, 1), lambda n, r: (n, 0)),
        ],
        out_shape=[
            jax.ShapeDtypeStruct((R, N, D), jnp.float32),
            jax.ShapeDtypeStruct((N, R), jnp.float32),
            jax.ShapeDtypeStruct((N, R), jnp.float32),
            jax.ShapeDtypeStruct((N, 1), jnp.float32),
        ],
    )(x, W, Bs, Bd)


def _norm_body(o0_ref, o1_ref, d0_ref, d1_ref, x_ref, *, relu):
    o = o0_ref[0] + o1_ref[0]
    d = d0_ref[0] + d1_ref[0]
    x = o * (1.0 / (d + 1e-16))
    if relu:
        x = jnp.maximum(x, 0.0)
    x_ref[...] = x


def _normalize(outp, denp, relu):
    denp3 = denp.reshape(NC, NSH, 1)
    return pl.pallas_call(
        functools.partial(_norm_body, relu=relu),
        grid=(NB,),
        in_specs=[
            pl.BlockSpec((1, TN, D), lambda n: (0, n, 0)),
            pl.BlockSpec((1, TN, D), lambda n: (1, n, 0)),
            pl.BlockSpec((1, TN, 1), lambda n: (0, n, 0)),
            pl.BlockSpec((1, TN, 1), lambda n: (1, n, 0)),
        ],
        out_specs=pl.BlockSpec((TN, D), lambda n: (n, 0)),
        out_shape=jax.ShapeDtypeStruct((N, D), jnp.float32),
    )(outp, outp, denp3, denp3)


def _sc_body(ih_hbm, is_hbm, id_hbm, dst_hbm, ssrc_hbm, sdst_hbm, cmax_hbm,
             h_hbm, outp_hbm, denp_hbm,
             ih0, is0, id0, dst0, dsc0, es0, ed0, cb0, ex0, rows0,
             ih1, is1, id1, dst1, dsc1, es1, ed1, cb1, ex1, rows1,
             zden_v, out_sh, den_sh,
             semlin0, semg0, semsc0, semlin1, semg1, semsc1):
    cid = lax.axis_index("c")
    sid = lax.axis_index("s")
    wid = sid * NC + cid

    slots = (
        (ih0, is0, id0, dst0, dsc0, es0, ed0, cb0, ex0, rows0,
         semlin0, semg0, semsc0),
        (ih1, is1, id1, dst1, dsc1, es1, ed1, cb1, ex1, rows1,
         semlin1, semg1, semsc1),
    )

    def lin_issue(ci, s):
        ih, is_, id_, dst = s[0], s[1], s[2], s[3]
        semlin = s[10]
        base = pl.multiple_of(wid * EW + ci * K, K)
        pltpu.async_copy(ih_hbm.at[pl.ds(base, K)], ih, semlin)
        pltpu.async_copy(is_hbm.at[pl.ds(base, K)], is_, semlin)
        pltpu.async_copy(id_hbm.at[pl.ds(base, K)], id_, semlin)
        pltpu.async_copy(dst_hbm.at[pl.ds(base, K)], dst, semlin)

    def lin_wait(s):
        ih, is_, id_, dst = s[0], s[1], s[2], s[3]
        semlin = s[10]
        zk = pl.ds(0, K)
        pltpu.make_async_copy(ih_hbm.at[zk], ih, semlin).wait()
        pltpu.make_async_copy(is_hbm.at[zk], is_, semlin).wait()
        pltpu.make_async_copy(id_hbm.at[zk], id_, semlin).wait()
        pltpu.make_async_copy(dst_hbm.at[zk], dst, semlin).wait()

    def gather_issue(s):
        ih, is_, id_, dst = s[0], s[1], s[2], s[3]
        es, ed, cb, rows = s[5], s[6], s[7], s[9]
        semg = s[11]
        pltpu.async_copy(ssrc_hbm.at[is_], es, semg)
        pltpu.async_copy(sdst_hbm.at[id_], ed, semg)
        pltpu.async_copy(cmax_hbm.at[dst], cb, semg)
        pltpu.async_copy(h_hbm.at[ih], rows, semg)

    def gather_wait(s):
        ih, is_, id_, dst = s[0], s[1], s[2], s[3]
        es, ed, cb, rows = s[5], s[6], s[7], s[9]
        semg = s[11]
        pltpu.make_async_copy(ssrc_hbm.at[is_], es, semg).wait()
        pltpu.make_async_copy(sdst_hbm.at[id_], ed, semg).wait()
        pltpu.make_async_copy(cmax_hbm.at[dst], cb, semg).wait()
        pltpu.make_async_copy(h_hbm.at[ih], rows, semg).wait()

    def scatter_issue(s):
        dsc, ex, rows, semsc = s[4], s[8], s[9], s[12]
        pltpu.async_copy(rows, out_sh.at[dsc], semsc, add=True)
        pltpu.async_copy(ex, den_sh.at[dsc], semsc, add=True)

    def scatter_wait(s):
        dsc, ex, rows, semsc = s[4], s[8], s[9], s[12]
        pltpu.make_async_copy(rows, out_sh.at[dsc], semsc).wait()
        pltpu.make_async_copy(ex, den_sh.at[dsc], semsc).wait()

    def compute(s):
        dst, dsc, es, ed, cb, ex, rows = s[3], s[4], s[5], s[6], s[7], s[8], s[9]
        for g in range(K // 16):
            sl = pl.ds(g * 16, 16)
            dsc[sl] = dst[sl]

        @pl.loop(0, K // 16)
        def _(g):
            sl = pl.ds(g * 16, 16)
            e = es[sl] + ed[sl]
            e = jnp.where(e >= 0.0, e, e * 0.2)
            ex[sl] = jnp.exp(e - cb[sl])

        @pl.loop(0, K)
        def _(i):
            sv = plsc.load_gather(ex, [jnp.full((16,), i, jnp.int32)])
            for j in range(D // 16):
                sl = pl.ds(j * 16, 16)
                rows[i, sl] = rows[i, sl] * sv

    @pl.loop(0, K)
    def _(i):
        for j in range(D // 16):
            rows0[i, pl.ds(j * 16, 16)] = jnp.zeros((16,), jnp.float32)

    @pl.loop(0, 1280 // 16)
    def _(i):
        zden_v[pl.ds(i * 16, 16)] = jnp.zeros((16,), jnp.float32)

    for z in range(ROWS_PER_SUB // K):
        pltpu.sync_copy(rows0, out_sh.at[pl.ds(sid * ROWS_PER_SUB + z * K, K)])

    @pl.when(sid == 0)
    def _():
        for z in range(NSH // 1280):
            pltpu.sync_copy(zden_v, den_sh.at[pl.ds(z * 1280, 1280)])

    plsc.subcore_barrier()

    lin_issue(0, slots[0])
    lin_wait(slots[0])
    gather_issue(slots[0])
    lin_issue(1, slots[1])

    @pl.loop(0, NCH)
    def _(ci):
        def body(s, o):
            @pl.when(ci <= NCH - 2)
            def _():
                lin_wait(o)

                @pl.when(ci >= 1)
                def _():
                    scatter_wait(o)

                gather_issue(o)
            gather_wait(s)

            @pl.when(ci <= NCH - 3)
            def _():
                lin_issue(ci + 2, s)
            compute(s)
            scatter_issue(s)

        @pl.when(ci % 2 == 0)
        def _():
            body(slots[0], slots[1])

        @pl.when(ci % 2 == 1)
        def _():
            body(slots[1], slots[0])

    scatter_wait(slots[(NCH - 1) % 2])
    scatter_wait(slots[NCH % 2])

    plsc.subcore_barrier()

    for z in range(ROWS_PER_SUB // K):
        off = sid * ROWS_PER_SUB + z * K
        pltpu.sync_copy(out_sh.at[pl.ds(off, K)], outp_hbm.at[cid, pl.ds(off, K)])

    @pl.when(sid == 0)
    def _():
        pltpu.sync_copy(den_sh, denp_hbm.at[cid])


_sc_edge = pl.kernel(
    _sc_body,
    out_type=[
        jax.ShapeDtypeStruct((NC, NSH, D), jnp.float32),
        jax.ShapeDtypeStruct((NC, NSH), jnp.float32),
    ],
    mesh=plsc.VectorSubcoreMesh(core_axis_name="c", subcore_axis_name="s"),
    compiler_params=pltpu.CompilerParams(needs_layout_passes=False),
    scratch_types=(
        [pltpu.VMEM((K,), jnp.int32)] * 5
        + [pltpu.VMEM((K,), jnp.float32)] * 4
        + [pltpu.VMEM((K, D), jnp.float32)]
    ) * 2 + [
        pltpu.VMEM((1280,), jnp.float32),
        pltpu.VMEM_SHARED((NSH, D), jnp.float32),
        pltpu.VMEM_SHARED((NSH,), jnp.float32),
        pltpu.SemaphoreType.DMA,
        pltpu.SemaphoreType.DMA,
        pltpu.SemaphoreType.DMA,
        pltpu.SemaphoreType.DMA,
        pltpu.SemaphoreType.DMA,
        pltpu.SemaphoreType.DMA,
    ],
)


def _layer(x, W, a_s, a_d, idx_h, idx_s, idx_d, dst_p):
    Bs = jnp.einsum('rdo,ro->dr', W, a_s)
    Bd = jnp.einsum('rdo,ro->dr', W, a_d)
    h, ssrc, sdst, rowmax = _dense(x, W, Bs, Bd)
    m_src = jnp.max(ssrc)
    cmax = jnp.concatenate(
        [jnp.maximum(rowmax[:, 0] + m_src, 0.0),
         jnp.full((1,), 1e30, jnp.float32)])
    outp, denp = _sc_edge(
        idx_h, idx_s, idx_d, dst_p,
        ssrc.reshape(N * R), sdst.reshape(N * R), cmax,
        h.reshape(R * N, D))
    return outp, denp


def kernel(x, edge_index, edge_type, W0, a_src0, a_dst0, W1, a_src1, a_dst1,
           W2, a_src2, a_dst2):
    src = edge_index[0]
    dst = edge_index[1]
    pad = EP - E
    src_p = jnp.concatenate([src, jnp.zeros((pad,), jnp.int32)])
    dst_p = jnp.concatenate([dst, jnp.full((pad,), N, jnp.int32)])
    et_p = jnp.concatenate([edge_type, jnp.zeros((pad,), jnp.int32)])
    idx_h, idx_s, idx_d = _prep_indices(src_p, dst_p, et_p)

    outp, denp = _layer(x, W0, a_src0, a_dst0, idx_h, idx_s, idx_d, dst_p)
    x1 = _normalize(outp, denp, relu=True)
    outp, denp = _layer(x1, W1, a_src1, a_dst1, idx_h, idx_s, idx_d, dst_p)
    x2 = _normalize(outp, denp, relu=True)
    outp, denp = _layer(x2, W2, a_src2, a_dst2, idx_h, idx_s, idx_d, dst_p)
    return _normalize(outp, denp, relu=False)


# speedup vs baseline: 13.5756x; 1.0071x over previous
import functools

import jax
import jax.numpy as jnp
from jax import lax
from jax.experimental import pallas as pl
from jax.experimental.pallas import tpu as pltpu
from jax.experimental.pallas import tpu_sc as plsc

N = 10000
E = 160000
R = 8
D = 128

NC = 2
NS = 16
NW = NC * NS
K = 128
EW = 5120
NCH = EW // K
EP = NW * EW
NSH = 10240
ROWS_PER_SUB = NSH // NS

TN = 400
NB = N // TN


def _prep_body(s_ref, d_ref, t_ref, ih_ref, is_ref, id_ref):
    s = s_ref[...]
    d = d_ref[...]
    t = t_ref[...]
    ih_ref[...] = t * N + s
    is_ref[...] = s * R + t
    id_ref[...] = jnp.minimum(d, N - 1) * R + t


def _prep_indices(src_p, dst_p, et_p):
    shp = (EP // 128, 128)
    out = pl.pallas_call(
        _prep_body,
        out_shape=[jax.ShapeDtypeStruct(shp, jnp.int32)] * 3,
    )(src_p.reshape(shp), dst_p.reshape(shp), et_p.reshape(shp))
    return [o.reshape(EP) for o in out]


def _dense_body(x_ref, w_ref, bs_ref, bd_ref,
                h_ref, ss_ref, sd_ref, rm_ref):
    xb = x_ref[...]
    h_ref[0] = jnp.dot(xb, w_ref[0], preferred_element_type=jnp.float32)

    @pl.when(pl.program_id(1) == 0)
    def _():
        ss = jnp.dot(xb, bs_ref[...], preferred_element_type=jnp.float32)
        sd = jnp.dot(xb, bd_ref[...], preferred_element_type=jnp.float32)
        ss_ref[...] = ss
        sd_ref[...] = sd
        rm_ref[...] = jnp.max(sd, axis=1, keepdims=True)


def _dense(x, W, Bs, Bd):
    return pl.pallas_call(
        _dense_body,
        grid=(NB, R),
        in_specs=[
            pl.BlockSpec((TN, D), lambda n, r: (n, 0)),
            pl.BlockSpec((1, D, D), lambda n, r: (r, 0, 0)),
            pl.BlockSpec((D, R), lambda n, r: (0, 0)),
            pl.BlockSpec((D, R), lambda n, r: (0, 0)),
        ],
        out_specs=[
            pl.BlockSpec((1, TN, D), lambda n, r: (r, n, 0)),
            pl.BlockSpec((TN, R), lambda n, r: (n, 0)),
            pl.BlockSpec((TN, R), lambda n, r: (n, 0)),
            pl.BlockSpec((TN, 1), lambda n, r: (n, 0)),
        ],
        out_shape=[
            jax.ShapeDtypeStruct((R, N, D), jnp.float32),
            jax.ShapeDtypeStruct((N, R), jnp.float32),
            jax.ShapeDtypeStruct((N, R), jnp.float32),
            jax.ShapeDtypeStruct((N, 1), jnp.float32),
        ],
    )(x, W, Bs, Bd)


def _dense2_body(o0_ref, o1_ref, d0_ref, d1_ref, w_ref, bs_ref, bd_ref,
                 h_ref, ss_ref, sd_ref, rm_ref, x_s):
    @pl.when(pl.program_id(1) == 0)
    def _():
        o = o0_ref[0] + o1_ref[0]
        d = d0_ref[0] + d1_ref[0]
        xb = jnp.maximum(o * (1.0 / (d + 1e-16)), 0.0)
        x_s[...] = xb
        ss = jnp.dot(xb, bs_ref[...], preferred_element_type=jnp.float32)
        sd = jnp.dot(xb, bd_ref[...], preferred_element_type=jnp.float32)
        ss_ref[...] = ss
        sd_ref[...] = sd
        rm_ref[...] = jnp.max(sd, axis=1, keepdims=True)

    h_ref[0] = jnp.dot(x_s[...], w_ref[0], preferred_element_type=jnp.float32)


def _dense2(outp, denp, W, Bs, Bd):
    denp3 = denp.reshape(NC, NSH, 1)
    return pl.pallas_call(
        _dense2_body,
        grid=(NB, R),
        in_specs=[
            pl.BlockSpec((1, TN, D), lambda n, r: (0, n, 0)),
            pl.BlockSpec((1, TN, D), lambda n, r: (1, n, 0)),
            pl.BlockSpec((1, TN, 1), lambda n, r: (0, n, 0)),
            pl.BlockSpec((1, TN, 1), lambda n, r: (1, n, 0)),
            pl.BlockSpec((1, D, D), lambda n, r: (r, 0, 0)),
            pl.BlockSpec((D, R), lambda n, r: (0, 0)),
            pl.BlockSpec((D, R), lambda n, r: (0, 0)),
        ],
        out_specs=[
            pl.BlockSpec((1, TN, D), lambda n, r: (r, n, 0)),
            pl.BlockSpec((TN, R), lambda n, r: (n, 0)),
            pl.BlockSpec((TN, R), lambda n, r: (n, 0)),
            pl.BlockSpec((TN, 1), lambda n, r: (n, 0)),
        ],
        out_shape=[
            jax.ShapeDtypeStruct((R, N, D), jnp.float32),
            jax.ShapeDtypeStruct((N, R), jnp.float32),
            jax.ShapeDtypeStruct((N, R), jnp.float32),
            jax.ShapeDtypeStruct((N, 1), jnp.float32),
        ],
        scratch_shapes=[pltpu.VMEM((TN, D), jnp.float32)],
    )(outp, outp, denp3, denp3, W, Bs, Bd)


def _norm_body(o0_ref, o1_ref, d0_ref, d1_ref, x_ref, *, relu):
    o = o0_ref[0] + o1_ref[0]
    d = d0_ref[0] + d1_ref[0]
    x = o * (1.0 / (d + 1e-16))
    if relu:
        x = jnp.maximum(x, 0.0)
    x_ref[...] = x


def _normalize(outp, denp, relu):
    denp3 = denp.reshape(NC, NSH, 1)
    return pl.pallas_call(
        functools.partial(_norm_body, relu=relu),
        grid=(NB,),
        in_specs=[
            pl.BlockSpec((1, TN, D), lambda n: (0, n, 0)),
            pl.BlockSpec((1, TN, D), lambda n: (1, n, 0)),
            pl.BlockSpec((1, TN, 1), lambda n: (0, n, 0)),
            pl.BlockSpec((1, TN, 1), lambda n: (1, n, 0)),
        ],
        out_specs=pl.BlockSpec((TN, D), lambda n: (n, 0)),
        out_shape=jax.ShapeDtypeStruct((N, D), jnp.float32),
    )(outp, outp, denp3, denp3)


def _sc_body(ih_hbm, is_hbm, id_hbm, dst_hbm, ssrc_hbm, sdst_hbm, cmax_hbm,
             h_hbm, outp_hbm, denp_hbm,
             ih0, is0, id0, dst0, dsc0, es0, ed0, cb0, ex0, rows0,
             ih1, is1, id1, dst1, dsc1, es1, ed1, cb1, ex1, rows1,
             zden_v, out_sh, den_sh,
             semlin0, semg0, semsc0, semlin1, semg1, semsc1):
    cid = lax.axis_index("c")
    sid = lax.axis_index("s")
    wid = sid * NC + cid

    slots = (
        (ih0, is0, id0, dst0, dsc0, es0, ed0, cb0, ex0, rows0,
         semlin0, semg0, semsc0),
        (ih1, is1, id1, dst1, dsc1, es1, ed1, cb1, ex1, rows1,
         semlin1, semg1, semsc1),
    )

    def lin_issue(ci, s):
        ih, is_, id_, dst = s[0], s[1], s[2], s[3]
        semlin = s[10]
        base = pl.multiple_of(wid * EW + ci * K, K)
        pltpu.async_copy(ih_hbm.at[pl.ds(base, K)], ih, semlin)
        pltpu.async_copy(is_hbm.at[pl.ds(base, K)], is_, semlin)
        pltpu.async_copy(id_hbm.at[pl.ds(base, K)], id_, semlin)
        pltpu.async_copy(dst_hbm.at[pl.ds(base, K)], dst, semlin)

    def lin_wait(s):
        ih, is_, id_, dst = s[0], s[1], s[2], s[3]
        semlin = s[10]
        zk = pl.ds(0, K)
        pltpu.make_async_copy(ih_hbm.at[zk], ih, semlin).wait()
        pltpu.make_async_copy(is_hbm.at[zk], is_, semlin).wait()
        pltpu.make_async_copy(id_hbm.at[zk], id_, semlin).wait()
        pltpu.make_async_copy(dst_hbm.at[zk], dst, semlin).wait()

    def gather_issue(s):
        ih, is_, id_, dst = s[0], s[1], s[2], s[3]
        es, ed, cb, rows = s[5], s[6], s[7], s[9]
        semg = s[11]
        pltpu.async_copy(ssrc_hbm.at[is_], es, semg)
        pltpu.async_copy(sdst_hbm.at[id_], ed, semg)
        pltpu.async_copy(cmax_hbm.at[dst], cb, semg)
        pltpu.async_copy(h_hbm.at[ih], rows, semg)

    def gather_wait(s):
        ih, is_, id_, dst = s[0], s[1], s[2], s[3]
        es, ed, cb, rows = s[5], s[6], s[7], s[9]
        semg = s[11]
        pltpu.make_async_copy(ssrc_hbm.at[is_], es, semg).wait()
        pltpu.make_async_copy(sdst_hbm.at[id_], ed, semg).wait()
        pltpu.make_async_copy(cmax_hbm.at[dst], cb, semg).wait()
        pltpu.make_async_copy(h_hbm.at[ih], rows, semg).wait()

    def scatter_issue(s):
        dsc, ex, rows, semsc = s[4], s[8], s[9], s[12]
        pltpu.async_copy(rows, out_sh.at[dsc], semsc, add=True)
        pltpu.async_copy(ex, den_sh.at[dsc], semsc, add=True)

    def scatter_wait(s):
        dsc, ex, rows, semsc = s[4], s[8], s[9], s[12]
        pltpu.make_async_copy(rows, out_sh.at[dsc], semsc).wait()
        pltpu.make_async_copy(ex, den_sh.at[dsc], semsc).wait()

    def compute(s):
        dst, dsc, es, ed, cb, ex, rows = s[3], s[4], s[5], s[6], s[7], s[8], s[9]
        for g in range(K // 16):
            sl = pl.ds(g * 16, 16)
            dsc[sl] = dst[sl]

        @pl.loop(0, K // 16)
        def _(g):
            sl = pl.ds(g * 16, 16)
            e = es[sl] + ed[sl]
            e = jnp.where(e >= 0.0, e, e * 0.2)
            ex[sl] = jnp.exp(e - cb[sl])

        @pl.loop(0, K)
        def _(i):
            sv = plsc.load_gather(ex, [jnp.full((16,), i, jnp.int32)])
            for j in range(D // 16):
                sl = pl.ds(j * 16, 16)
                rows[i, sl] = rows[i, sl] * sv

    @pl.loop(0, K)
    def _(i):
        for j in range(D // 16):
            rows0[i, pl.ds(j * 16, 16)] = jnp.zeros((16,), jnp.float32)

    @pl.loop(0, 1280 // 16)
    def _(i):
        zden_v[pl.ds(i * 16, 16)] = jnp.zeros((16,), jnp.float32)

    for z in range(ROWS_PER_SUB // K):
        pltpu.sync_copy(rows0, out_sh.at[pl.ds(sid * ROWS_PER_SUB + z * K, K)])

    @pl.when(sid == 0)
    def _():
        for z in range(NSH // 1280):
            pltpu.sync_copy(zden_v, den_sh.at[pl.ds(z * 1280, 1280)])

    plsc.subcore_barrier()

    lin_issue(0, slots[0])
    lin_wait(slots[0])
    gather_issue(slots[0])
    lin_issue(1, slots[1])

    @pl.loop(0, NCH)
    def _(ci):
        def body(s, o):
            @pl.when(ci <= NCH - 2)
            def _():
                lin_wait(o)

                @pl.when(ci >= 1)
                def _():
                    scatter_wait(o)

                gather_issue(o)
            gather_wait(s)

            @pl.when(ci <= NCH - 3)
            def _():
                lin_issue(ci + 2, s)
            compute(s)
            scatter_issue(s)

        @pl.when(ci % 2 == 0)
        def _():
            body(slots[0], slots[1])

        @pl.when(ci % 2 == 1)
        def _():
            body(slots[1], slots[0])

    scatter_wait(slots[(NCH - 1) % 2])
    scatter_wait(slots[NCH % 2])

    plsc.subcore_barrier()

    for z in range(ROWS_PER_SUB // K):
        off = sid * ROWS_PER_SUB + z * K
        pltpu.sync_copy(out_sh.at[pl.ds(off, K)], outp_hbm.at[cid, pl.ds(off, K)])

    @pl.when(sid == 0)
    def _():
        pltpu.sync_copy(den_sh, denp_hbm.at[cid])


_sc_edge = pl.kernel(
    _sc_body,
    out_type=[
        jax.ShapeDtypeStruct((NC, NSH, D), jnp.float32),
        jax.ShapeDtypeStruct((NC, NSH), jnp.float32),
    ],
    mesh=plsc.VectorSubcoreMesh(core_axis_name="c", subcore_axis_name="s"),
    compiler_params=pltpu.CompilerParams(needs_layout_passes=False),
    scratch_types=(
        [pltpu.VMEM((K,), jnp.int32)] * 5
        + [pltpu.VMEM((K,), jnp.float32)] * 4
        + [pltpu.VMEM((K, D), jnp.float32)]
    ) * 2 + [
        pltpu.VMEM((1280,), jnp.float32),
        pltpu.VMEM_SHARED((NSH, D), jnp.float32),
        pltpu.VMEM_SHARED((NSH,), jnp.float32),
        pltpu.SemaphoreType.DMA,
        pltpu.SemaphoreType.DMA,
        pltpu.SemaphoreType.DMA,
        pltpu.SemaphoreType.DMA,
        pltpu.SemaphoreType.DMA,
        pltpu.SemaphoreType.DMA,
    ],
)


def _layer(x, W, a_s, a_d, idx_h, idx_s, idx_d, dst_p):
    Bs = jnp.einsum('rdo,ro->dr', W, a_s)
    Bd = jnp.einsum('rdo,ro->dr', W, a_d)
    if isinstance(x, tuple):
        h, ssrc, sdst, rowmax = _dense2(x[0], x[1], W, Bs, Bd)
    else:
        h, ssrc, sdst, rowmax = _dense(x, W, Bs, Bd)
    m_src = jnp.max(ssrc)
    cmax = jnp.concatenate(
        [jnp.maximum(rowmax[:, 0] + m_src, 0.0),
         jnp.full((1,), 1e30, jnp.float32)])
    outp, denp = _sc_edge(
        idx_h, idx_s, idx_d, dst_p,
        ssrc.reshape(N * R), sdst.reshape(N * R), cmax,
        h.reshape(R * N, D))
    return outp, denp


def kernel(x, edge_index, edge_type, W0, a_src0, a_dst0, W1, a_src1, a_dst1,
           W2, a_src2, a_dst2):
    src = edge_index[0]
    dst = edge_index[1]
    pad = EP - E
    src_p = jnp.concatenate([src, jnp.zeros((pad,), jnp.int32)])
    dst_p = jnp.concatenate([dst, jnp.full((pad,), N, jnp.int32)])
    et_p = jnp.concatenate([edge_type, jnp.zeros((pad,), jnp.int32)])
    idx_h, idx_s, idx_d = _prep_indices(src_p, dst_p, et_p)

    outp, denp = _layer(x, W0, a_src0, a_dst0, idx_h, idx_s, idx_d, dst_p)
    outp, denp = _layer((outp, denp), W1, a_src1, a_dst1,
                        idx_h, idx_s, idx_d, dst_p)
    outp, denp = _layer((outp, denp), W2, a_src2, a_dst2,
                        idx_h, idx_s, idx_d, dst_p)
    return _normalize(outp, denp, relu=False)
